# baseline (speedup 1.0000x reference)
import functools

import jax
import jax.numpy as jnp
from jax import lax
from jax.experimental import pallas as pl
from jax.experimental.pallas import tpu as pltpu
from jax.experimental.pallas import tpu_sc as plsc

N = 10000
E = 320000
B = 64
S = 256
D = 128
NC = 2
NS = 16
NPAD = 10240
CH = 125
NCHUNK = E // CH
W_CHUNKS = NCHUNK // (NC * NS)
HALF = W_CHUNKS // 2
T_NODES = NPAD // NS

_mesh = plsc.VectorSubcoreMesh(core_axis_name="c", subcore_axis_name="s")


def _prep_body(table, dst3d, bid3d, loc3d, bid6, zconst, onesc,
               x_out, degp_out, cnt_out,
               dbuf, bbuf, lbuf, fbuf, rowbuf, onesv, cbuf6, acc, acc2,
               asem, gsem, wsem):
    c = lax.axis_index("c")
    s = lax.axis_index("s")
    wid = s * NC + c

    pltpu.sync_copy(zconst, acc.at[pl.ds(s * T_NODES, T_NODES)])
    pltpu.sync_copy(onesc, onesv)

    @pl.when(c == 0)
    def _cnt_zero():
        pltpu.sync_copy(zconst.at[pl.ds(0, 8)], acc2.at[pl.ds(s * 8, 8)])
        pltpu.sync_copy(bid6.at[s], cbuf6)

    @pl.when(c == 1)
    def _flatidx():
        pltpu.sync_copy(bid3d.at[s], bbuf)
        pltpu.sync_copy(loc3d.at[s], lbuf)

        def _flat(i, _):
            r = i // 8
            cs = (i % 8) * 16
            bv = bbuf[r, pl.ds(cs, 16)]
            lv = lbuf[r, pl.ds(cs, 16)]
            fbuf[r, pl.ds(cs, 16)] = jnp.minimum(bv * S + lv, B * S - 1)
            return _
        lax.fori_loop(0, T_NODES // 16, _flat, None)

    plsc.subcore_barrier()

    @pl.when(c == 0)
    def _cnt():
        for k in range(6):
            pltpu.sync_copy(onesv, acc2.at[cbuf6.at[k]], add=True)

    for k in range(5):
        @pl.when(c == 1)
        def _g_start():
            if k > 0:
                pltpu.make_async_copy(
                    rowbuf, x_out.at[pl.ds(s * T_NODES + (k - 1) * 128, 128)],
                    wsem).wait()
            pltpu.async_copy(table.at[fbuf.at[k]], rowbuf, gsem)

        pltpu.sync_copy(dst3d.at[wid].at[pl.ds(k * 16, 16)], dbuf)

        def _grp(g, _):
            for b in range(4):
                pltpu.async_copy(onesv, acc.at[dbuf.at[g * 4 + b]], asem,
                                 add=True)
            for b in range(4):
                pltpu.make_async_copy(onesv, acc.at[dbuf.at[g * 4 + b]],
                                      asem).wait()
            return _
        lax.fori_loop(0, 4, _grp, None)

        @pl.when(c == 1)
        def _g_drain():
            pltpu.make_async_copy(table.at[fbuf.at[k]], rowbuf, gsem).wait()
            pltpu.async_copy(rowbuf,
                             x_out.at[pl.ds(s * T_NODES + k * 128, 128)], wsem)

    @pl.when(c == 1)
    def _w_last():
        pltpu.make_async_copy(
            rowbuf, x_out.at[pl.ds(s * T_NODES + 4 * 128, 128)], wsem).wait()
    plsc.subcore_barrier()
    pltpu.sync_copy(acc.at[pl.ds(s * T_NODES, T_NODES)],
                    degp_out.at[c].at[pl.ds(s * T_NODES, T_NODES)])

    @pl.when((c == 0) & (s < 8))
    def _cnt_out():
        pltpu.sync_copy(acc2.at[pl.ds(s * 8, 8)], cnt_out.at[pl.ds(s * 8, 8)])


_sc_prep = functools.partial(
    pl.kernel,
    out_type=[
        jax.ShapeDtypeStruct((NPAD, D), jnp.float32),
        jax.ShapeDtypeStruct((NC, NPAD, D), jnp.float32),
        jax.ShapeDtypeStruct((B, D), jnp.float32),
    ],
    mesh=_mesh,
    scratch_types=[
        pltpu.VMEM((16, CH), jnp.int32),
        pltpu.VMEM((5, 128), jnp.int32),
        pltpu.VMEM((5, 128), jnp.int32),
        pltpu.VMEM((5, 128), jnp.int32),
        pltpu.VMEM((128, D), jnp.float32),
        pltpu.VMEM((CH, D), jnp.float32),
        pltpu.VMEM((6, CH), jnp.int32),
        pltpu.VMEM_SHARED((NPAD, D), jnp.float32),
        pltpu.VMEM_SHARED((128, 128), jnp.float32),
        pltpu.SemaphoreType.DMA,
        pltpu.SemaphoreType.DMA,
        pltpu.SemaphoreType.DMA,
    ],
)(_prep_body)


def _agg_body(hp, src3d, dst3d, zconst, agg_out,
              sbuf, dbuf, rbuf, acc, *sems):
    c = lax.axis_index("c")
    s = lax.axis_index("s")
    gsem = sems[:2]
    ssem = sems[2:]
    wid = s * NC + c

    pltpu.sync_copy(zconst, acc.at[pl.ds(s * T_NODES, T_NODES)])
    plsc.subcore_barrier()

    for half in range(2):
        pltpu.sync_copy(src3d.at[wid].at[pl.ds(half * HALF, HALF)], sbuf)
        pltpu.sync_copy(dst3d.at[wid].at[pl.ds(half * HALF, HALF)], dbuf)

        def _g_start(j, b):
            pltpu.async_copy(hp.at[sbuf.at[j].at[pl.ds(0, 64)]],
                             rbuf.at[pl.ds(b * CH, 64)], gsem[b])
            pltpu.async_copy(hp.at[sbuf.at[j].at[pl.ds(64, CH - 64)]],
                             rbuf.at[pl.ds(b * CH + 64, CH - 64)], gsem[b])

        def _g_wait(j, b):
            pltpu.make_async_copy(hp.at[sbuf.at[j].at[pl.ds(0, 64)]],
                                  rbuf.at[pl.ds(b * CH, 64)], gsem[b]).wait()
            pltpu.make_async_copy(hp.at[sbuf.at[j].at[pl.ds(64, CH - 64)]],
                                  rbuf.at[pl.ds(b * CH + 64, CH - 64)],
                                  gsem[b]).wait()

        for b in range(2):
            _g_start(b, b)

        def _step(jo, _):
            for b in range(2):
                j = jo * 2 + b
                slot = rbuf.at[pl.ds(b * CH, CH)]
                _g_wait(j, b)
                pltpu.async_copy(slot, acc.at[dbuf.at[j]], ssem[b], add=True)
                pltpu.make_async_copy(slot, acc.at[dbuf.at[j]], ssem[b]).wait()

                @pl.when(j + 2 < HALF)
                def _():
                    _g_start(j + 2, b)
            return _
        lax.fori_loop(0, HALF // 2, _step, None)
    plsc.subcore_barrier()
    pltpu.sync_copy(acc.at[pl.ds(s * T_NODES, T_NODES)],
                    agg_out.at[c].at[pl.ds(s * T_NODES, T_NODES)])


_sc_agg = functools.partial(
    pl.kernel,
    out_type=jax.ShapeDtypeStruct((NC, NPAD, D), jnp.float32),
    mesh=_mesh,
    scratch_types=[
        pltpu.VMEM((HALF, CH), jnp.int32),
        pltpu.VMEM((HALF, CH), jnp.int32),
        pltpu.VMEM((2 * CH, D), jnp.float32),
        pltpu.VMEM_SHARED((NPAD, D), jnp.float32),
        pltpu.SemaphoreType.DMA,
        pltpu.SemaphoreType.DMA,
        pltpu.SemaphoreType.DMA,
        pltpu.SemaphoreType.DMA,
    ],
)(_agg_body)


def _pool_body(tmp, bid3d, cnt, zconst, gout,
               qbuf, vbuf, obuf, cbuf, acc):
    c = lax.axis_index("c")
    s = lax.axis_index("s")

    @pl.when(c == 0)
    def _():
        pltpu.sync_copy(bid3d.at[s], qbuf)
        pltpu.sync_copy(zconst.at[pl.ds(0, 8)], acc.at[pl.ds(s * 8, 8)])
        plsc.subcore_barrier()
        for k in range(5):
            pltpu.sync_copy(tmp.at[pl.ds(s * T_NODES + k * 128, 128)], vbuf)
            pltpu.sync_copy(vbuf, acc.at[qbuf.at[k]], add=True)
        plsc.subcore_barrier()

        @pl.when(s < 8)
        def _out():
            pltpu.sync_copy(acc.at[pl.ds(s * 8, 8)], obuf)
            pltpu.sync_copy(cnt.at[pl.ds(s * 8, 8)], cbuf)
            for r in range(8):
                cv = cbuf[r, pl.ds(0, 16)]
                inv = 1.0 / jnp.maximum(cv, 1.0)
                for k in range(8):
                    obuf[r, pl.ds(k * 16, 16)] = obuf[r, pl.ds(k * 16, 16)] * inv
            pltpu.sync_copy(obuf, gout.at[pl.ds(s * 8, 8)])


_sc_pool = functools.partial(
    pl.kernel,
    out_type=jax.ShapeDtypeStruct((B, D), jnp.float32),
    mesh=_mesh,
    scratch_types=[
        pltpu.VMEM((5, 128), jnp.int32),
        pltpu.VMEM((128, D), jnp.float32),
        pltpu.VMEM((8, D), jnp.float32),
        pltpu.VMEM((8, D), jnp.float32),
        pltpu.VMEM_SHARED((128, 128), jnp.float32),
    ],
)(_pool_body)


_R = 2048


def _b1_body(x, w, d0, d1, o):
    dinv = lax.rsqrt(d0[:, :1] + d1[:, :1] + 1.0)
    o[...] = jnp.dot(x[...], w[...], preferred_element_type=jnp.float32) * dinv


def _tc_mm1(x, w, d0, d1):
    return pl.pallas_call(
        _b1_body,
        grid=(NPAD // _R,),
        in_specs=[
            pl.BlockSpec((_R, D), lambda i: (i, 0)),
            pl.BlockSpec((D, D), lambda i: (0, 0)),
            pl.BlockSpec((_R, D), lambda i: (i, 0)),
            pl.BlockSpec((_R, D), lambda i: (i, 0)),
        ],
        out_specs=pl.BlockSpec((_R, D), lambda i: (i, 0)),
        out_shape=jax.ShapeDtypeStruct((NPAD, D), jnp.float32),
    )(x, w, d0, d1)


def _b2_body(a0, a1, hp, d0, d1, b1r, w, o):
    dinv = lax.rsqrt(d0[:, :1] + d1[:, :1] + 1.0)
    x1 = jnp.maximum(dinv * (a0[...] + a1[...] + hp[...]) + b1r[...], 0.0)
    o[...] = jnp.dot(x1, w[...], preferred_element_type=jnp.float32) * dinv


def _tc_mm2(a0, a1, hp, d0, d1, b1r, w):
    return pl.pallas_call(
        _b2_body,
        grid=(NPAD // _R,),
        in_specs=[
            pl.BlockSpec((_R, D), lambda i: (i, 0)),
            pl.BlockSpec((_R, D), lambda i: (i, 0)),
            pl.BlockSpec((_R, D), lambda i: (i, 0)),
            pl.BlockSpec((_R, D), lambda i: (i, 0)),
            pl.BlockSpec((_R, D), lambda i: (i, 0)),
            pl.BlockSpec((1, D), lambda i: (0, 0)),
            pl.BlockSpec((D, D), lambda i: (0, 0)),
        ],
        out_specs=pl.BlockSpec((_R, D), lambda i: (i, 0)),
        out_shape=jax.ShapeDtypeStruct((NPAD, D), jnp.float32),
    )(a0, a1, hp, d0, d1, b1r, w)


def _b3_body(a0, a1, hp, d0, d1, b2r, o):
    dinv = lax.rsqrt(d0[:, :1] + d1[:, :1] + 1.0)
    o[...] = dinv * (a0[...] + a1[...] + hp[...]) + b2r[...]


def _tc_fin(a0, a1, hp, d0, d1, b2r):
    return pl.pallas_call(
        _b3_body,
        grid=(NPAD // _R,),
        in_specs=[
            pl.BlockSpec((_R, D), lambda i: (i, 0)),
            pl.BlockSpec((_R, D), lambda i: (i, 0)),
            pl.BlockSpec((_R, D), lambda i: (i, 0)),
            pl.BlockSpec((_R, D), lambda i: (i, 0)),
            pl.BlockSpec((_R, D), lambda i: (i, 0)),
            pl.BlockSpec((1, D), lambda i: (0, 0)),
        ],
        out_specs=pl.BlockSpec((_R, D), lambda i: (i, 0)),
        out_shape=jax.ShapeDtypeStruct((NPAD, D), jnp.float32),
    )(a0, a1, hp, d0, d1, b2r)


def kernel(drug_graph_embedding, edge_index, batch_ids, local_indices,
           W1, b1, W2, b2):
    table = drug_graph_embedding.reshape(B * S, D)
    src3d = edge_index[0].reshape(NC * NS, W_CHUNKS, CH)
    dst3d = edge_index[1].reshape(NC * NS, W_CHUNKS, CH)
    pad = NPAD - N
    bid3d = jnp.concatenate(
        [batch_ids, jnp.full((pad,), B, jnp.int32)]).reshape(NS, 5, 128)
    loc3d = jnp.concatenate(
        [local_indices, jnp.zeros((pad,), jnp.int32)]).reshape(NS, 5, 128)
    bid6 = jnp.concatenate(
        [batch_ids, jnp.full((2000,), B, jnp.int32)])[:NS * 6 * CH]
    bid6 = bid6.reshape(NS, 6, CH)
    zconst = jnp.zeros((T_NODES, 128), jnp.float32)
    onesc = jnp.ones((CH, D), jnp.float32)

    x_pad, degp, cnt = _sc_prep(table, dst3d, bid3d, loc3d, bid6, zconst, onesc)
    d0 = degp[0]
    d1 = degp[1]

    hp1 = _tc_mm1(x_pad, W1, d0, d1)
    agg1 = _sc_agg(hp1, src3d, dst3d, zconst)
    hp2 = _tc_mm2(agg1[0], agg1[1], hp1, d0, d1, b1.reshape(1, D), W2)
    agg2 = _sc_agg(hp2, src3d, dst3d, zconst)
    tmp = _tc_fin(agg2[0], agg2[1], hp2, d0, d1, b2.reshape(1, D))
    return _sc_pool(tmp, bid3d, cnt, zconst)


# speedup vs baseline: 1.0042x; 1.0042x over previous
import functools

import jax
import jax.numpy as jnp
from jax import lax
from jax.experimental import pallas as pl
from jax.experimental.pallas import tpu as pltpu
from jax.experimental.pallas import tpu_sc as plsc

N = 10000
E = 320000
B = 64
S = 256
D = 128
NC = 2
NS = 16
NPAD = 10240
CH = 125
NCHUNK = E // CH
W_CHUNKS = NCHUNK // (NC * NS)
HALF = W_CHUNKS // 2
T_NODES = NPAD // NS

_mesh = plsc.VectorSubcoreMesh(core_axis_name="c", subcore_axis_name="s")


def _prep_body(table, dst3d, bid3d, loc3d, bid6, zconst, onesc,
               x_out, degp_out, cnt_out,
               dbuf, bbuf, lbuf, fbuf, rowbuf, onesv, cbuf6, acc, acc2,
               asem, gsem, wsem):
    c = lax.axis_index("c")
    s = lax.axis_index("s")
    wid = s * NC + c

    pltpu.sync_copy(zconst, acc.at[pl.ds(s * T_NODES, T_NODES)])
    pltpu.sync_copy(onesc, onesv)

    @pl.when(c == 0)
    def _cnt_zero():
        pltpu.sync_copy(zconst.at[pl.ds(0, 8)], acc2.at[pl.ds(s * 8, 8)])
        pltpu.sync_copy(bid6.at[s], cbuf6)

    @pl.when(c == 1)
    def _flatidx():
        pltpu.sync_copy(bid3d.at[s], bbuf)
        pltpu.sync_copy(loc3d.at[s], lbuf)

        def _flat(i, _):
            r = i // 8
            cs = (i % 8) * 16
            bv = bbuf[r, pl.ds(cs, 16)]
            lv = lbuf[r, pl.ds(cs, 16)]
            fbuf[r, pl.ds(cs, 16)] = jnp.minimum(bv * S + lv, B * S - 1)
            return _
        lax.fori_loop(0, T_NODES // 16, _flat, None)

    plsc.subcore_barrier()

    @pl.when(c == 0)
    def _cnt():
        for k in range(6):
            pltpu.sync_copy(onesv, acc2.at[cbuf6.at[k]], add=True)

    for k in range(5):
        @pl.when(c == 1)
        def _g_start():
            if k > 0:
                pltpu.make_async_copy(
                    rowbuf, x_out.at[pl.ds(s * T_NODES + (k - 1) * 128, 128)],
                    wsem).wait()
            pltpu.async_copy(table.at[fbuf.at[k]], rowbuf, gsem)

        pltpu.sync_copy(dst3d.at[wid].at[pl.ds(k * 16, 16)], dbuf)

        def _grp(g, _):
            for b in range(4):
                pltpu.async_copy(onesv, acc.at[dbuf.at[g * 4 + b]], asem,
                                 add=True)
            for b in range(4):
                pltpu.make_async_copy(onesv, acc.at[dbuf.at[g * 4 + b]],
                                      asem).wait()
            return _
        lax.fori_loop(0, 4, _grp, None)

        @pl.when(c == 1)
        def _g_drain():
            pltpu.make_async_copy(table.at[fbuf.at[k]], rowbuf, gsem).wait()
            pltpu.async_copy(rowbuf,
                             x_out.at[pl.ds(s * T_NODES + k * 128, 128)], wsem)

    @pl.when(c == 1)
    def _w_last():
        pltpu.make_async_copy(
            rowbuf, x_out.at[pl.ds(s * T_NODES + 4 * 128, 128)], wsem).wait()
    plsc.subcore_barrier()
    pltpu.sync_copy(acc.at[pl.ds(s * T_NODES, T_NODES)],
                    degp_out.at[c].at[pl.ds(s * T_NODES, T_NODES)])

    @pl.when((c == 0) & (s < 8))
    def _cnt_out():
        pltpu.sync_copy(acc2.at[pl.ds(s * 8, 8)], cnt_out.at[pl.ds(s * 8, 8)])


_sc_prep = functools.partial(
    pl.kernel,
    out_type=[
        jax.ShapeDtypeStruct((NPAD, D), jnp.float32),
        jax.ShapeDtypeStruct((NC, NPAD, D), jnp.float32),
        jax.ShapeDtypeStruct((B, D), jnp.float32),
    ],
    mesh=_mesh,
    scratch_types=[
        pltpu.VMEM((16, CH), jnp.int32),
        pltpu.VMEM((5, 128), jnp.int32),
        pltpu.VMEM((5, 128), jnp.int32),
        pltpu.VMEM((5, 128), jnp.int32),
        pltpu.VMEM((128, D), jnp.float32),
        pltpu.VMEM((CH, D), jnp.float32),
        pltpu.VMEM((6, CH), jnp.int32),
        pltpu.VMEM_SHARED((NPAD, D), jnp.float32),
        pltpu.VMEM_SHARED((128, 128), jnp.float32),
        pltpu.SemaphoreType.DMA,
        pltpu.SemaphoreType.DMA,
        pltpu.SemaphoreType.DMA,
    ],
)(_prep_body)


def _agg_body(hp, src3d, dst3d, zconst, agg_out,
              sbuf, dbuf, rbuf, acc, *sems):
    c = lax.axis_index("c")
    s = lax.axis_index("s")
    gsem = sems[:2]
    ssem = sems[2:]
    wid = s * NC + c

    pltpu.sync_copy(zconst, acc.at[pl.ds(s * T_NODES, T_NODES)])
    plsc.subcore_barrier()

    for half in range(2):
        pltpu.sync_copy(src3d.at[wid].at[pl.ds(half * HALF, HALF)], sbuf)
        pltpu.sync_copy(dst3d.at[wid].at[pl.ds(half * HALF, HALF)], dbuf)

        def _g_start(j, b):
            pltpu.async_copy(hp.at[sbuf.at[j].at[pl.ds(0, 64)]],
                             rbuf.at[pl.ds(b * CH, 64)], gsem[b])
            pltpu.async_copy(hp.at[sbuf.at[j].at[pl.ds(64, CH - 64)]],
                             rbuf.at[pl.ds(b * CH + 64, CH - 64)], gsem[b])

        def _g_wait(j, b):
            pltpu.make_async_copy(hp.at[sbuf.at[j].at[pl.ds(0, 64)]],
                                  rbuf.at[pl.ds(b * CH, 64)], gsem[b]).wait()
            pltpu.make_async_copy(hp.at[sbuf.at[j].at[pl.ds(64, CH - 64)]],
                                  rbuf.at[pl.ds(b * CH + 64, CH - 64)],
                                  gsem[b]).wait()

        for b in range(2):
            _g_start(b, b)

        def _step(jo, _):
            for b in range(2):
                j = jo * 2 + b
                slot = rbuf.at[pl.ds(b * CH, CH)]
                _g_wait(j, b)
                pltpu.async_copy(slot, acc.at[dbuf.at[j]], ssem[b], add=True)
                pltpu.make_async_copy(slot, acc.at[dbuf.at[j]], ssem[b]).wait()

                @pl.when(j + 2 < HALF)
                def _():
                    _g_start(j + 2, b)
            return _
        lax.fori_loop(0, HALF // 2, _step, None)
    plsc.subcore_barrier()
    pltpu.sync_copy(acc.at[pl.ds(s * T_NODES, T_NODES)],
                    agg_out.at[c].at[pl.ds(s * T_NODES, T_NODES)])


_sc_agg = functools.partial(
    pl.kernel,
    out_type=jax.ShapeDtypeStruct((NC, NPAD, D), jnp.float32),
    mesh=_mesh,
    scratch_types=[
        pltpu.VMEM((HALF, CH), jnp.int32),
        pltpu.VMEM((HALF, CH), jnp.int32),
        pltpu.VMEM((2 * CH, D), jnp.float32),
        pltpu.VMEM_SHARED((NPAD, D), jnp.float32),
        pltpu.SemaphoreType.DMA,
        pltpu.SemaphoreType.DMA,
        pltpu.SemaphoreType.DMA,
        pltpu.SemaphoreType.DMA,
    ],
)(_agg_body)


def _pool_body(tmp, bid3d, cnt, zconst, gout,
               qbuf, vbuf, obuf, cbuf, acc):
    c = lax.axis_index("c")
    s = lax.axis_index("s")

    @pl.when(c == 0)
    def _():
        pltpu.sync_copy(bid3d.at[s], qbuf)
        pltpu.sync_copy(zconst.at[pl.ds(0, 8)], acc.at[pl.ds(s * 8, 8)])
        plsc.subcore_barrier()
        for k in range(5):
            pltpu.sync_copy(tmp.at[pl.ds(s * T_NODES + k * 128, 128)], vbuf)
            pltpu.sync_copy(vbuf, acc.at[qbuf.at[k]], add=True)
        plsc.subcore_barrier()

        @pl.when(s < 8)
        def _out():
            pltpu.sync_copy(acc.at[pl.ds(s * 8, 8)], obuf)
            pltpu.sync_copy(cnt.at[pl.ds(s * 8, 8)], cbuf)
            for r in range(8):
                cv = cbuf[r, pl.ds(0, 16)]
                inv = 1.0 / jnp.maximum(cv, 1.0)
                for k in range(8):
                    obuf[r, pl.ds(k * 16, 16)] = obuf[r, pl.ds(k * 16, 16)] * inv
            pltpu.sync_copy(obuf, gout.at[pl.ds(s * 8, 8)])


_sc_pool = functools.partial(
    pl.kernel,
    out_type=jax.ShapeDtypeStruct((B, D), jnp.float32),
    mesh=_mesh,
    scratch_types=[
        pltpu.VMEM((5, 128), jnp.int32),
        pltpu.VMEM((128, D), jnp.float32),
        pltpu.VMEM((8, D), jnp.float32),
        pltpu.VMEM((8, D), jnp.float32),
        pltpu.VMEM_SHARED((128, 128), jnp.float32),
    ],
)(_pool_body)


_R = 2048


def _b1_body(x, w, d0, d1, o, dv):
    dinv = lax.rsqrt(d0[:, :1] + d1[:, :1] + 1.0)
    dv[...] = dinv
    o[...] = jnp.dot(x[...], w[...], preferred_element_type=jnp.float32) * dinv


def _tc_mm1(x, w, d0, d1):
    return pl.pallas_call(
        _b1_body,
        grid=(NPAD // _R,),
        in_specs=[
            pl.BlockSpec((_R, D), lambda i: (i, 0)),
            pl.BlockSpec((D, D), lambda i: (0, 0)),
            pl.BlockSpec((_R, D), lambda i: (i, 0)),
            pl.BlockSpec((_R, D), lambda i: (i, 0)),
        ],
        out_specs=[
            pl.BlockSpec((_R, D), lambda i: (i, 0)),
            pl.BlockSpec((_R, 1), lambda i: (i, 0)),
        ],
        out_shape=[
            jax.ShapeDtypeStruct((NPAD, D), jnp.float32),
            jax.ShapeDtypeStruct((NPAD, 1), jnp.float32),
        ],
    )(x, w, d0, d1)


def _b2_body(a0, a1, hp, dv, b1r, w, o):
    dinv = dv[...]
    x1 = jnp.maximum(dinv * (a0[...] + a1[...] + hp[...]) + b1r[...], 0.0)
    o[...] = jnp.dot(x1, w[...], preferred_element_type=jnp.float32) * dinv


def _tc_mm2(a0, a1, hp, dv, b1r, w):
    return pl.pallas_call(
        _b2_body,
        grid=(NPAD // _R,),
        in_specs=[
            pl.BlockSpec((_R, D), lambda i: (i, 0)),
            pl.BlockSpec((_R, D), lambda i: (i, 0)),
            pl.BlockSpec((_R, D), lambda i: (i, 0)),
            pl.BlockSpec((_R, 1), lambda i: (i, 0)),
            pl.BlockSpec((1, D), lambda i: (0, 0)),
            pl.BlockSpec((D, D), lambda i: (0, 0)),
        ],
        out_specs=pl.BlockSpec((_R, D), lambda i: (i, 0)),
        out_shape=jax.ShapeDtypeStruct((NPAD, D), jnp.float32),
    )(a0, a1, hp, dv, b1r, w)


def _b3_body(a0, a1, hp, dv, b2r, o):
    o[...] = dv[...] * (a0[...] + a1[...] + hp[...]) + b2r[...]


def _tc_fin(a0, a1, hp, dv, b2r):
    return pl.pallas_call(
        _b3_body,
        grid=(NPAD // _R,),
        in_specs=[
            pl.BlockSpec((_R, D), lambda i: (i, 0)),
            pl.BlockSpec((_R, D), lambda i: (i, 0)),
            pl.BlockSpec((_R, D), lambda i: (i, 0)),
            pl.BlockSpec((_R, 1), lambda i: (i, 0)),
            pl.BlockSpec((1, D), lambda i: (0, 0)),
        ],
        out_specs=pl.BlockSpec((_R, D), lambda i: (i, 0)),
        out_shape=jax.ShapeDtypeStruct((NPAD, D), jnp.float32),
    )(a0, a1, hp, dv, b2r)


def kernel(drug_graph_embedding, edge_index, batch_ids, local_indices,
           W1, b1, W2, b2):
    table = drug_graph_embedding.reshape(B * S, D)
    src3d = edge_index[0].reshape(NC * NS, W_CHUNKS, CH)
    dst3d = edge_index[1].reshape(NC * NS, W_CHUNKS, CH)
    pad = NPAD - N
    bid3d = jnp.concatenate(
        [batch_ids, jnp.full((pad,), B, jnp.int32)]).reshape(NS, 5, 128)
    loc3d = jnp.concatenate(
        [local_indices, jnp.zeros((pad,), jnp.int32)]).reshape(NS, 5, 128)
    bid6 = jnp.concatenate(
        [batch_ids, jnp.full((2000,), B, jnp.int32)])[:NS * 6 * CH]
    bid6 = bid6.reshape(NS, 6, CH)
    zconst = jnp.zeros((T_NODES, 128), jnp.float32)
    onesc = jnp.ones((CH, D), jnp.float32)

    x_pad, degp, cnt = _sc_prep(table, dst3d, bid3d, loc3d, bid6, zconst, onesc)
    d0 = degp[0]
    d1 = degp[1]

    hp1, dinvcol = _tc_mm1(x_pad, W1, d0, d1)
    agg1 = _sc_agg(hp1, src3d, dst3d, zconst)
    hp2 = _tc_mm2(agg1[0], agg1[1], hp1, dinvcol, b1.reshape(1, D), W2)
    agg2 = _sc_agg(hp2, src3d, dst3d, zconst)
    tmp = _tc_fin(agg2[0], agg2[1], hp2, dinvcol, b2.reshape(1, D))
    return _sc_pool(tmp, bid3d, cnt, zconst)
